# Initial kernel scaffold; baseline (speedup 1.0000x reference)
import jax
import jax.numpy as jnp
from jax.experimental import pallas as pl


def kernel(x, W, top_k):
    raise NotImplementedError("write your pallas kernel here")


# speedup vs baseline: 1.4644x; 1.4644x over previous
import jax
import jax.numpy as jnp
from jax.experimental import pallas as pl

_HIDDEN = 4096
_EXPERTS = 64
_BT = 512


def _gating_body(x_ref, w_ref, wout_ref, iout_ref):
    logits = jax.lax.dot_general(
        x_ref[...], w_ref[...],
        (((1,), (1,)), ((), ())),
        preferred_element_type=jnp.float32,
    )
    lane = jax.lax.broadcasted_iota(jnp.int32, logits.shape, 1)
    m1 = jnp.max(logits, axis=1, keepdims=True)
    i1 = jnp.min(jnp.where(logits == m1, lane, _EXPERTS), axis=1, keepdims=True)
    masked = jnp.where(lane == i1, -jnp.inf, logits)
    m2 = jnp.max(masked, axis=1, keepdims=True)
    i2 = jnp.min(jnp.where(masked == m2, lane, _EXPERTS), axis=1, keepdims=True)
    e2 = jnp.exp(m2 - m1)
    denom = 1.0 + e2
    w1 = 1.0 / denom
    w2 = e2 / denom
    wout_ref[...] = jnp.concatenate([w1, w2], axis=1)
    iout_ref[...] = jnp.concatenate([i1, i2], axis=1)


def kernel(x, W, top_k):
    b, s, h = x.shape
    tokens = b * s
    x2 = x.reshape(tokens, h)
    grid = (tokens // _BT,)
    wout, iout = pl.pallas_call(
        _gating_body,
        grid=grid,
        in_specs=[
            pl.BlockSpec((_BT, h), lambda i: (i, 0)),
            pl.BlockSpec((_EXPERTS, h), lambda i: (0, 0)),
        ],
        out_specs=[
            pl.BlockSpec((_BT, 2), lambda i: (i, 0)),
            pl.BlockSpec((_BT, 2), lambda i: (i, 0)),
        ],
        out_shape=[
            jax.ShapeDtypeStruct((tokens, 2), jnp.float32),
            jax.ShapeDtypeStruct((tokens, 2), jnp.int32),
        ],
    )(x2, W)
    return wout.reshape(b, s, 2), iout.reshape(b, s, 2)


# speedup vs baseline: 1.5363x; 1.0490x over previous
import jax
import jax.numpy as jnp
from jax.experimental import pallas as pl

_HIDDEN = 4096
_EXPERTS = 64
_BT = 1024


def _gating_body(x_ref, w_ref, wout_ref, iout_ref):
    logits = jax.lax.dot_general(
        x_ref[...], w_ref[...],
        (((1,), (1,)), ((), ())),
        preferred_element_type=jnp.float32,
    )
    lane = jax.lax.broadcasted_iota(jnp.int32, logits.shape, 1)
    m1 = jnp.max(logits, axis=1, keepdims=True)
    i1 = jnp.min(jnp.where(logits == m1, lane, _EXPERTS), axis=1, keepdims=True)
    masked = jnp.where(lane == i1, -jnp.inf, logits)
    m2 = jnp.max(masked, axis=1, keepdims=True)
    i2 = jnp.min(jnp.where(masked == m2, lane, _EXPERTS), axis=1, keepdims=True)
    e2 = jnp.exp(m2 - m1)
    denom = 1.0 + e2
    w1 = 1.0 / denom
    w2 = e2 / denom
    wout_ref[...] = jnp.concatenate([w1, w2], axis=1)
    iout_ref[...] = jnp.concatenate([i1, i2], axis=1)


def kernel(x, W, top_k):
    b, s, h = x.shape
    tokens = b * s
    x2 = x.reshape(tokens, h)
    grid = (tokens // _BT,)
    wout, iout = pl.pallas_call(
        _gating_body,
        grid=grid,
        in_specs=[
            pl.BlockSpec((_BT, h), lambda i: (i, 0)),
            pl.BlockSpec((_EXPERTS, h), lambda i: (0, 0)),
        ],
        out_specs=[
            pl.BlockSpec((_BT, 2), lambda i: (i, 0)),
            pl.BlockSpec((_BT, 2), lambda i: (i, 0)),
        ],
        out_shape=[
            jax.ShapeDtypeStruct((tokens, 2), jnp.float32),
            jax.ShapeDtypeStruct((tokens, 2), jnp.int32),
        ],
    )(x2, W)
    return wout.reshape(b, s, 2), iout.reshape(b, s, 2)
